# baseline (speedup 1.0000x reference)
import functools

import jax
import jax.numpy as jnp
from jax import lax
from jax.experimental import pallas as pl
from jax.experimental.pallas import tpu as pltpu
from jax.experimental.pallas import tpu_sc as plsc

_NUM_CORES = 2
_NUM_SUBCORES = 16
_NUM_WORKERS = _NUM_CORES * _NUM_SUBCORES


@functools.partial(jax.jit, static_argnums=(1, 2))
def _sc_slice_copy(pe2d, seq_len, d_model):
    rows_per_w = seq_len // _NUM_WORKERS
    mesh = plsc.VectorSubcoreMesh(
        core_axis_name="c", subcore_axis_name="s")

    n_chunks = 4
    chunk = rows_per_w // n_chunks

    @functools.partial(
        pl.kernel,
        mesh=mesh,
        out_type=jax.ShapeDtypeStruct((seq_len, d_model), pe2d.dtype),
        scratch_types=[
            pltpu.VMEM((2, chunk, d_model), pe2d.dtype),
            pltpu.SemaphoreType.DMA,
            pltpu.SemaphoreType.DMA,
        ],
    )
    def copy_kernel(pe_hbm, out_hbm, buf, in_sem, out_sem):
        wid = lax.axis_index("s") * _NUM_CORES + lax.axis_index("c")
        base = wid * rows_per_w

        loads = []
        stores = []
        for i in range(n_chunks):
            lo = base + i * chunk
            loads.append(pltpu.make_async_copy(
                pe_hbm.at[pl.ds(lo, chunk)], buf.at[i % 2], in_sem))
            stores.append(pltpu.make_async_copy(
                buf.at[i % 2], out_hbm.at[pl.ds(lo, chunk)], out_sem))

        loads[0].start()
        loads[1].start()
        for i in range(n_chunks):
            loads[i].wait()
            stores[i].start()
            if i + 2 < n_chunks:
                stores[i].wait()
                loads[i + 2].start()
        stores[n_chunks - 2].wait()
        stores[n_chunks - 1].wait()

    return copy_kernel(pe2d)


@functools.partial(jax.jit, static_argnums=(1, 2, 3))
def _sc_direct_copy(pe2d, seq_len, d_model, n_chunks=1):
    rows_per_w = seq_len // _NUM_WORKERS
    chunk = rows_per_w // n_chunks
    mesh = plsc.VectorSubcoreMesh(
        core_axis_name="c", subcore_axis_name="s")

    @functools.partial(
        pl.kernel,
        mesh=mesh,
        out_type=jax.ShapeDtypeStruct((seq_len, d_model), pe2d.dtype),
        scratch_types=[pltpu.SemaphoreType.DMA],
    )
    def copy_kernel(pe_hbm, out_hbm, sem):
        wid = lax.axis_index("s") * _NUM_CORES + lax.axis_index("c")
        base = wid * rows_per_w
        copies = []
        for i in range(n_chunks):
            lo = base + i * chunk
            copies.append(pltpu.make_async_copy(
                pe_hbm.at[pl.ds(lo, chunk)], out_hbm.at[pl.ds(lo, chunk)],
                sem))
        for c in copies:
            c.start()
        for c in copies:
            c.wait()

    return copy_kernel(pe2d)


def _tc_direct_copy(pe2d, seq_len, d_model, n_chunks=8):
    chunk = seq_len // n_chunks

    def body(src_hbm, out_hbm, sem):
        copies = []
        for i in range(n_chunks):
            lo = i * chunk
            copies.append(pltpu.make_async_copy(
                src_hbm.at[pl.ds(lo, chunk)], out_hbm.at[pl.ds(lo, chunk)],
                sem))
        for c in copies:
            c.start()
        for c in copies:
            c.wait()

    return pl.pallas_call(
        body,
        in_specs=[pl.BlockSpec(memory_space=pltpu.MemorySpace.HBM)],
        out_specs=pl.BlockSpec(memory_space=pltpu.MemorySpace.HBM),
        out_shape=jax.ShapeDtypeStruct((seq_len, d_model), pe2d.dtype),
        scratch_shapes=[pltpu.SemaphoreType.DMA],
    )(pe2d)


def _tc_dma_copy(pe2d, seq_len, d_model, n_chunks=8, nbuf=4):
    chunk = seq_len // n_chunks

    assert nbuf == n_chunks

    def body(src_hbm, out_hbm, buf, in_sems, out_sem):
        loads = []
        stores = []
        for i in range(n_chunks):
            lo = i * chunk
            loads.append(pltpu.make_async_copy(
                src_hbm.at[pl.ds(lo, chunk)], buf.at[i], in_sems.at[i]))
            stores.append(pltpu.make_async_copy(
                buf.at[i], out_hbm.at[pl.ds(lo, chunk)], out_sem))
        for i in range(n_chunks):
            loads[i].start()
        for i in range(n_chunks):
            loads[i].wait()
            stores[i].start()
        for i in range(n_chunks):
            stores[i].wait()

    return pl.pallas_call(
        body,
        in_specs=[pl.BlockSpec(memory_space=pltpu.MemorySpace.HBM)],
        out_specs=pl.BlockSpec(memory_space=pltpu.MemorySpace.HBM),
        out_shape=jax.ShapeDtypeStruct((seq_len, d_model), pe2d.dtype),
        scratch_shapes=[
            pltpu.VMEM((nbuf, chunk, d_model), pe2d.dtype),
            pltpu.SemaphoreType.DMA((n_chunks,)),
            pltpu.SemaphoreType.DMA,
        ],
    )(pe2d)


def kernel(a_x, pe):
    seq_len = a_x.shape[1]
    _, max_len, d_model = pe.shape
    out = _sc_direct_copy(pe.reshape(max_len, d_model), seq_len, d_model)
    return out.reshape(1, seq_len, d_model)


# speedup vs baseline: 1.0634x; 1.0634x over previous
import functools

import jax
import jax.numpy as jnp
from jax import lax
from jax.experimental import pallas as pl
from jax.experimental.pallas import tpu as pltpu
from jax.experimental.pallas import tpu_sc as plsc

_NUM_CORES = 2
_NUM_SUBCORES = 16
_NUM_WORKERS = _NUM_CORES * _NUM_SUBCORES


@functools.partial(jax.jit, static_argnums=(1, 2))
def _sc_slice_copy(pe2d, seq_len, d_model):
    rows_per_w = seq_len // _NUM_WORKERS
    mesh = plsc.VectorSubcoreMesh(
        core_axis_name="c", subcore_axis_name="s")

    n_chunks = 4
    chunk = rows_per_w // n_chunks

    @functools.partial(
        pl.kernel,
        mesh=mesh,
        out_type=jax.ShapeDtypeStruct((seq_len, d_model), pe2d.dtype),
        scratch_types=[
            pltpu.VMEM((2, chunk, d_model), pe2d.dtype),
            pltpu.SemaphoreType.DMA,
            pltpu.SemaphoreType.DMA,
        ],
    )
    def copy_kernel(pe_hbm, out_hbm, buf, in_sem, out_sem):
        wid = lax.axis_index("s") * _NUM_CORES + lax.axis_index("c")
        base = wid * rows_per_w

        loads = []
        stores = []
        for i in range(n_chunks):
            lo = base + i * chunk
            loads.append(pltpu.make_async_copy(
                pe_hbm.at[pl.ds(lo, chunk)], buf.at[i % 2], in_sem))
            stores.append(pltpu.make_async_copy(
                buf.at[i % 2], out_hbm.at[pl.ds(lo, chunk)], out_sem))

        loads[0].start()
        loads[1].start()
        for i in range(n_chunks):
            loads[i].wait()
            stores[i].start()
            if i + 2 < n_chunks:
                stores[i].wait()
                loads[i + 2].start()
        stores[n_chunks - 2].wait()
        stores[n_chunks - 1].wait()

    return copy_kernel(pe2d)


@functools.partial(jax.jit, static_argnums=(1, 2, 3))
def _sc_direct_copy(pe2d, seq_len, d_model, n_chunks=1):
    rows_per_w = seq_len // _NUM_WORKERS
    chunk = rows_per_w // n_chunks
    mesh = plsc.VectorSubcoreMesh(
        core_axis_name="c", subcore_axis_name="s")

    @functools.partial(
        pl.kernel,
        mesh=mesh,
        out_type=jax.ShapeDtypeStruct((seq_len, d_model), pe2d.dtype),
        scratch_types=[pltpu.SemaphoreType.DMA],
    )
    def copy_kernel(pe_hbm, out_hbm, sem):
        wid = lax.axis_index("s") * _NUM_CORES + lax.axis_index("c")
        base = wid * rows_per_w
        copies = []
        for i in range(n_chunks):
            lo = base + i * chunk
            copies.append(pltpu.make_async_copy(
                pe_hbm.at[pl.ds(lo, chunk)], out_hbm.at[pl.ds(lo, chunk)],
                sem))
        for c in copies:
            c.start()
        for c in copies:
            c.wait()

    return copy_kernel(pe2d)


def _tc_direct_copy(pe2d, seq_len, d_model, n_chunks=8):
    chunk = seq_len // n_chunks

    def body(src_hbm, out_hbm, sem):
        copies = []
        for i in range(n_chunks):
            lo = i * chunk
            copies.append(pltpu.make_async_copy(
                src_hbm.at[pl.ds(lo, chunk)], out_hbm.at[pl.ds(lo, chunk)],
                sem))
        for c in copies:
            c.start()
        for c in copies:
            c.wait()

    return pl.pallas_call(
        body,
        in_specs=[pl.BlockSpec(memory_space=pltpu.MemorySpace.HBM)],
        out_specs=pl.BlockSpec(memory_space=pltpu.MemorySpace.HBM),
        out_shape=jax.ShapeDtypeStruct((seq_len, d_model), pe2d.dtype),
        scratch_shapes=[pltpu.SemaphoreType.DMA],
    )(pe2d)


def _tc_dma_copy(pe2d, seq_len, d_model, n_chunks=8, nbuf=4):
    chunk = seq_len // n_chunks

    assert nbuf == n_chunks

    def body(src_hbm, out_hbm, buf, in_sems, out_sem):
        loads = []
        stores = []
        for i in range(n_chunks):
            lo = i * chunk
            loads.append(pltpu.make_async_copy(
                src_hbm.at[pl.ds(lo, chunk)], buf.at[i], in_sems.at[i]))
            stores.append(pltpu.make_async_copy(
                buf.at[i], out_hbm.at[pl.ds(lo, chunk)], out_sem))
        for i in range(n_chunks):
            loads[i].start()
        for i in range(n_chunks):
            loads[i].wait()
            stores[i].start()
        for i in range(n_chunks):
            stores[i].wait()

    return pl.pallas_call(
        body,
        in_specs=[pl.BlockSpec(memory_space=pltpu.MemorySpace.HBM)],
        out_specs=pl.BlockSpec(memory_space=pltpu.MemorySpace.HBM),
        out_shape=jax.ShapeDtypeStruct((seq_len, d_model), pe2d.dtype),
        scratch_shapes=[
            pltpu.VMEM((nbuf, chunk, d_model), pe2d.dtype),
            pltpu.SemaphoreType.DMA((n_chunks,)),
            pltpu.SemaphoreType.DMA,
        ],
    )(pe2d)


def kernel(a_x, pe):
    seq_len = a_x.shape[1]
    _, max_len, d_model = pe.shape
    out = _tc_direct_copy(pe.reshape(max_len, d_model), seq_len, d_model,
                          n_chunks=8)
    return out.reshape(1, seq_len, d_model)


# speedup vs baseline: 41.4228x; 38.9527x over previous
import functools

import jax
import jax.numpy as jnp
from jax import lax
from jax.experimental import pallas as pl
from jax.experimental.pallas import tpu as pltpu
from jax.experimental.pallas import tpu_sc as plsc

_NUM_CORES = 2
_NUM_SUBCORES = 16
_NUM_WORKERS = _NUM_CORES * _NUM_SUBCORES


@functools.partial(jax.jit, static_argnums=(1, 2))
def _sc_slice_copy(pe2d, seq_len, d_model):
    rows_per_w = seq_len // _NUM_WORKERS
    mesh = plsc.VectorSubcoreMesh(
        core_axis_name="c", subcore_axis_name="s")

    n_chunks = 4
    chunk = rows_per_w // n_chunks

    @functools.partial(
        pl.kernel,
        mesh=mesh,
        out_type=jax.ShapeDtypeStruct((seq_len, d_model), pe2d.dtype),
        scratch_types=[
            pltpu.VMEM((2, chunk, d_model), pe2d.dtype),
            pltpu.SemaphoreType.DMA,
            pltpu.SemaphoreType.DMA,
        ],
    )
    def copy_kernel(pe_hbm, out_hbm, buf, in_sem, out_sem):
        wid = lax.axis_index("s") * _NUM_CORES + lax.axis_index("c")
        base = wid * rows_per_w

        loads = []
        stores = []
        for i in range(n_chunks):
            lo = base + i * chunk
            loads.append(pltpu.make_async_copy(
                pe_hbm.at[pl.ds(lo, chunk)], buf.at[i % 2], in_sem))
            stores.append(pltpu.make_async_copy(
                buf.at[i % 2], out_hbm.at[pl.ds(lo, chunk)], out_sem))

        loads[0].start()
        loads[1].start()
        for i in range(n_chunks):
            loads[i].wait()
            stores[i].start()
            if i + 2 < n_chunks:
                stores[i].wait()
                loads[i + 2].start()
        stores[n_chunks - 2].wait()
        stores[n_chunks - 1].wait()

    return copy_kernel(pe2d)


@functools.partial(jax.jit, static_argnums=(1, 2, 3))
def _sc_direct_copy(pe2d, seq_len, d_model, n_chunks=1):
    rows_per_w = seq_len // _NUM_WORKERS
    chunk = rows_per_w // n_chunks
    mesh = plsc.VectorSubcoreMesh(
        core_axis_name="c", subcore_axis_name="s")

    @functools.partial(
        pl.kernel,
        mesh=mesh,
        out_type=jax.ShapeDtypeStruct((seq_len, d_model), pe2d.dtype),
        scratch_types=[pltpu.SemaphoreType.DMA],
    )
    def copy_kernel(pe_hbm, out_hbm, sem):
        wid = lax.axis_index("s") * _NUM_CORES + lax.axis_index("c")
        base = wid * rows_per_w
        copies = []
        for i in range(n_chunks):
            lo = base + i * chunk
            copies.append(pltpu.make_async_copy(
                pe_hbm.at[pl.ds(lo, chunk)], out_hbm.at[pl.ds(lo, chunk)],
                sem))
        for c in copies:
            c.start()
        for c in copies:
            c.wait()

    return copy_kernel(pe2d)


def _tc_direct_copy(pe2d, seq_len, d_model, n_chunks=8):
    chunk = seq_len // n_chunks

    def body(src_hbm, out_hbm, sem):
        copies = []
        for i in range(n_chunks):
            lo = i * chunk
            copies.append(pltpu.make_async_copy(
                src_hbm.at[pl.ds(lo, chunk)], out_hbm.at[pl.ds(lo, chunk)],
                sem))
        for c in copies:
            c.start()
        for c in copies:
            c.wait()

    return pl.pallas_call(
        body,
        in_specs=[pl.BlockSpec(memory_space=pltpu.MemorySpace.HBM)],
        out_specs=pl.BlockSpec(memory_space=pltpu.MemorySpace.HBM),
        out_shape=jax.ShapeDtypeStruct((seq_len, d_model), pe2d.dtype),
        scratch_shapes=[pltpu.SemaphoreType.DMA],
    )(pe2d)


def _tc_dma_copy(pe2d, seq_len, d_model, n_chunks=8, nbuf=4):
    chunk = seq_len // n_chunks

    assert nbuf == n_chunks

    def body(src_hbm, out_hbm, buf, in_sems, out_sem):
        loads = []
        stores = []
        for i in range(n_chunks):
            lo = i * chunk
            loads.append(pltpu.make_async_copy(
                src_hbm.at[pl.ds(lo, chunk)], buf.at[i], in_sems.at[i]))
            stores.append(pltpu.make_async_copy(
                buf.at[i], out_hbm.at[pl.ds(lo, chunk)], out_sem))
        for i in range(n_chunks):
            loads[i].start()
        for i in range(n_chunks):
            loads[i].wait()
            stores[i].start()
        for i in range(n_chunks):
            stores[i].wait()

    return pl.pallas_call(
        body,
        in_specs=[pl.BlockSpec(memory_space=pltpu.MemorySpace.HBM)],
        out_specs=pl.BlockSpec(memory_space=pltpu.MemorySpace.HBM),
        out_shape=jax.ShapeDtypeStruct((seq_len, d_model), pe2d.dtype),
        scratch_shapes=[
            pltpu.VMEM((nbuf, chunk, d_model), pe2d.dtype),
            pltpu.SemaphoreType.DMA((n_chunks,)),
            pltpu.SemaphoreType.DMA,
        ],
    )(pe2d)


def kernel(a_x, pe):
    seq_len = a_x.shape[1]
    _, max_len, d_model = pe.shape
    out = _tc_dma_copy(pe.reshape(max_len, d_model), seq_len, d_model,
                       n_chunks=32, nbuf=32)
    return out.reshape(1, seq_len, d_model)


# speedup vs baseline: 42.8736x; 1.0350x over previous
import functools

import jax
import jax.numpy as jnp
from jax import lax
from jax.experimental import pallas as pl
from jax.experimental.pallas import tpu as pltpu
from jax.experimental.pallas import tpu_sc as plsc

_NUM_CORES = 2
_NUM_SUBCORES = 16
_NUM_WORKERS = _NUM_CORES * _NUM_SUBCORES


@functools.partial(jax.jit, static_argnums=(1, 2))
def _sc_slice_copy(pe2d, seq_len, d_model):
    rows_per_w = seq_len // _NUM_WORKERS
    mesh = plsc.VectorSubcoreMesh(
        core_axis_name="c", subcore_axis_name="s")

    n_chunks = 4
    chunk = rows_per_w // n_chunks

    @functools.partial(
        pl.kernel,
        mesh=mesh,
        out_type=jax.ShapeDtypeStruct((seq_len, d_model), pe2d.dtype),
        scratch_types=[
            pltpu.VMEM((2, chunk, d_model), pe2d.dtype),
            pltpu.SemaphoreType.DMA,
            pltpu.SemaphoreType.DMA,
        ],
    )
    def copy_kernel(pe_hbm, out_hbm, buf, in_sem, out_sem):
        wid = lax.axis_index("s") * _NUM_CORES + lax.axis_index("c")
        base = wid * rows_per_w

        loads = []
        stores = []
        for i in range(n_chunks):
            lo = base + i * chunk
            loads.append(pltpu.make_async_copy(
                pe_hbm.at[pl.ds(lo, chunk)], buf.at[i % 2], in_sem))
            stores.append(pltpu.make_async_copy(
                buf.at[i % 2], out_hbm.at[pl.ds(lo, chunk)], out_sem))

        loads[0].start()
        loads[1].start()
        for i in range(n_chunks):
            loads[i].wait()
            stores[i].start()
            if i + 2 < n_chunks:
                stores[i].wait()
                loads[i + 2].start()
        stores[n_chunks - 2].wait()
        stores[n_chunks - 1].wait()

    return copy_kernel(pe2d)


@functools.partial(jax.jit, static_argnums=(1, 2, 3))
def _sc_direct_copy(pe2d, seq_len, d_model, n_chunks=1):
    rows_per_w = seq_len // _NUM_WORKERS
    chunk = rows_per_w // n_chunks
    mesh = plsc.VectorSubcoreMesh(
        core_axis_name="c", subcore_axis_name="s")

    @functools.partial(
        pl.kernel,
        mesh=mesh,
        out_type=jax.ShapeDtypeStruct((seq_len, d_model), pe2d.dtype),
        scratch_types=[pltpu.SemaphoreType.DMA],
    )
    def copy_kernel(pe_hbm, out_hbm, sem):
        wid = lax.axis_index("s") * _NUM_CORES + lax.axis_index("c")
        base = wid * rows_per_w
        copies = []
        for i in range(n_chunks):
            lo = base + i * chunk
            copies.append(pltpu.make_async_copy(
                pe_hbm.at[pl.ds(lo, chunk)], out_hbm.at[pl.ds(lo, chunk)],
                sem))
        for c in copies:
            c.start()
        for c in copies:
            c.wait()

    return copy_kernel(pe2d)


def _tc_direct_copy(pe2d, seq_len, d_model, n_chunks=8):
    chunk = seq_len // n_chunks

    def body(src_hbm, out_hbm, sem):
        copies = []
        for i in range(n_chunks):
            lo = i * chunk
            copies.append(pltpu.make_async_copy(
                src_hbm.at[pl.ds(lo, chunk)], out_hbm.at[pl.ds(lo, chunk)],
                sem))
        for c in copies:
            c.start()
        for c in copies:
            c.wait()

    return pl.pallas_call(
        body,
        in_specs=[pl.BlockSpec(memory_space=pltpu.MemorySpace.HBM)],
        out_specs=pl.BlockSpec(memory_space=pltpu.MemorySpace.HBM),
        out_shape=jax.ShapeDtypeStruct((seq_len, d_model), pe2d.dtype),
        scratch_shapes=[pltpu.SemaphoreType.DMA],
    )(pe2d)


def _tc_dma_copy(pe2d, seq_len, d_model, n_chunks=8, nbuf=4):
    chunk = seq_len // n_chunks

    assert nbuf == n_chunks

    def body(src_hbm, out_hbm, buf, in_sems, out_sem):
        loads = []
        stores = []
        for i in range(n_chunks):
            lo = i * chunk
            loads.append(pltpu.make_async_copy(
                src_hbm.at[pl.ds(lo, chunk)], buf.at[i], in_sems.at[i]))
            stores.append(pltpu.make_async_copy(
                buf.at[i], out_hbm.at[pl.ds(lo, chunk)], out_sem))
        for i in range(n_chunks):
            loads[i].start()
        for i in range(n_chunks):
            loads[i].wait()
            stores[i].start()
        for i in range(n_chunks):
            stores[i].wait()

    return pl.pallas_call(
        body,
        in_specs=[pl.BlockSpec(memory_space=pltpu.MemorySpace.HBM)],
        out_specs=pl.BlockSpec(memory_space=pltpu.MemorySpace.HBM),
        out_shape=jax.ShapeDtypeStruct((seq_len, d_model), pe2d.dtype),
        scratch_shapes=[
            pltpu.VMEM((nbuf, chunk, d_model), pe2d.dtype),
            pltpu.SemaphoreType.DMA((n_chunks,)),
            pltpu.SemaphoreType.DMA,
        ],
    )(pe2d)


def kernel(a_x, pe):
    seq_len = a_x.shape[1]
    _, max_len, d_model = pe.shape
    out = _tc_dma_copy(pe.reshape(max_len, d_model), seq_len, d_model,
                       n_chunks=16, nbuf=16)
    return out.reshape(1, seq_len, d_model)


# speedup vs baseline: 43.8560x; 1.0229x over previous
import functools

import jax
import jax.numpy as jnp
from jax import lax
from jax.experimental import pallas as pl
from jax.experimental.pallas import tpu as pltpu
from jax.experimental.pallas import tpu_sc as plsc

_NUM_CORES = 2
_NUM_SUBCORES = 16
_NUM_WORKERS = _NUM_CORES * _NUM_SUBCORES


@functools.partial(jax.jit, static_argnums=(1, 2))
def _sc_slice_copy(pe2d, seq_len, d_model):
    rows_per_w = seq_len // _NUM_WORKERS
    mesh = plsc.VectorSubcoreMesh(
        core_axis_name="c", subcore_axis_name="s")

    n_chunks = 4
    chunk = rows_per_w // n_chunks

    @functools.partial(
        pl.kernel,
        mesh=mesh,
        out_type=jax.ShapeDtypeStruct((seq_len, d_model), pe2d.dtype),
        scratch_types=[
            pltpu.VMEM((2, chunk, d_model), pe2d.dtype),
            pltpu.SemaphoreType.DMA,
            pltpu.SemaphoreType.DMA,
        ],
    )
    def copy_kernel(pe_hbm, out_hbm, buf, in_sem, out_sem):
        wid = lax.axis_index("s") * _NUM_CORES + lax.axis_index("c")
        base = wid * rows_per_w

        loads = []
        stores = []
        for i in range(n_chunks):
            lo = base + i * chunk
            loads.append(pltpu.make_async_copy(
                pe_hbm.at[pl.ds(lo, chunk)], buf.at[i % 2], in_sem))
            stores.append(pltpu.make_async_copy(
                buf.at[i % 2], out_hbm.at[pl.ds(lo, chunk)], out_sem))

        loads[0].start()
        loads[1].start()
        for i in range(n_chunks):
            loads[i].wait()
            stores[i].start()
            if i + 2 < n_chunks:
                stores[i].wait()
                loads[i + 2].start()
        stores[n_chunks - 2].wait()
        stores[n_chunks - 1].wait()

    return copy_kernel(pe2d)


@functools.partial(jax.jit, static_argnums=(1, 2, 3))
def _sc_direct_copy(pe2d, seq_len, d_model, n_chunks=1):
    rows_per_w = seq_len // _NUM_WORKERS
    chunk = rows_per_w // n_chunks
    mesh = plsc.VectorSubcoreMesh(
        core_axis_name="c", subcore_axis_name="s")

    @functools.partial(
        pl.kernel,
        mesh=mesh,
        out_type=jax.ShapeDtypeStruct((seq_len, d_model), pe2d.dtype),
        scratch_types=[pltpu.SemaphoreType.DMA],
    )
    def copy_kernel(pe_hbm, out_hbm, sem):
        wid = lax.axis_index("s") * _NUM_CORES + lax.axis_index("c")
        base = wid * rows_per_w
        copies = []
        for i in range(n_chunks):
            lo = base + i * chunk
            copies.append(pltpu.make_async_copy(
                pe_hbm.at[pl.ds(lo, chunk)], out_hbm.at[pl.ds(lo, chunk)],
                sem))
        for c in copies:
            c.start()
        for c in copies:
            c.wait()

    return copy_kernel(pe2d)


def _tc_direct_copy(pe2d, seq_len, d_model, n_chunks=8):
    chunk = seq_len // n_chunks

    def body(src_hbm, out_hbm, sem):
        copies = []
        for i in range(n_chunks):
            lo = i * chunk
            copies.append(pltpu.make_async_copy(
                src_hbm.at[pl.ds(lo, chunk)], out_hbm.at[pl.ds(lo, chunk)],
                sem))
        for c in copies:
            c.start()
        for c in copies:
            c.wait()

    return pl.pallas_call(
        body,
        in_specs=[pl.BlockSpec(memory_space=pltpu.MemorySpace.HBM)],
        out_specs=pl.BlockSpec(memory_space=pltpu.MemorySpace.HBM),
        out_shape=jax.ShapeDtypeStruct((seq_len, d_model), pe2d.dtype),
        scratch_shapes=[pltpu.SemaphoreType.DMA],
    )(pe2d)


def _tc_dma_copy(pe2d, seq_len, d_model, n_chunks=8, nbuf=4):
    chunk = seq_len // n_chunks

    assert nbuf == n_chunks

    def body(src_hbm, out_hbm, buf, in_sems, out_sem):
        loads = []
        stores = []
        for i in range(n_chunks):
            lo = i * chunk
            loads.append(pltpu.make_async_copy(
                src_hbm.at[pl.ds(lo, chunk)], buf.at[i], in_sems.at[i]))
            stores.append(pltpu.make_async_copy(
                buf.at[i], out_hbm.at[pl.ds(lo, chunk)], out_sem))
        for i in range(n_chunks):
            loads[i].start()
        for i in range(n_chunks):
            loads[i].wait()
            stores[i].start()
        for i in range(n_chunks):
            stores[i].wait()

    return pl.pallas_call(
        body,
        in_specs=[pl.BlockSpec(memory_space=pltpu.MemorySpace.HBM)],
        out_specs=pl.BlockSpec(memory_space=pltpu.MemorySpace.HBM),
        out_shape=jax.ShapeDtypeStruct((seq_len, d_model), pe2d.dtype),
        scratch_shapes=[
            pltpu.VMEM((nbuf, chunk, d_model), pe2d.dtype),
            pltpu.SemaphoreType.DMA((n_chunks,)),
            pltpu.SemaphoreType.DMA,
        ],
    )(pe2d)


def kernel(a_x, pe):
    seq_len = a_x.shape[1]
    _, max_len, d_model = pe.shape
    out = _tc_dma_copy(pe.reshape(max_len, d_model), seq_len, d_model,
                       n_chunks=8, nbuf=8)
    return out.reshape(1, seq_len, d_model)


# speedup vs baseline: 44.0788x; 1.0051x over previous
import functools

import jax
import jax.numpy as jnp
from jax import lax
from jax.experimental import pallas as pl
from jax.experimental.pallas import tpu as pltpu
from jax.experimental.pallas import tpu_sc as plsc

_NUM_CORES = 2
_NUM_SUBCORES = 16
_NUM_WORKERS = _NUM_CORES * _NUM_SUBCORES


@functools.partial(jax.jit, static_argnums=(1, 2))
def _sc_slice_copy(pe2d, seq_len, d_model):
    rows_per_w = seq_len // _NUM_WORKERS
    mesh = plsc.VectorSubcoreMesh(
        core_axis_name="c", subcore_axis_name="s")

    n_chunks = 4
    chunk = rows_per_w // n_chunks

    @functools.partial(
        pl.kernel,
        mesh=mesh,
        out_type=jax.ShapeDtypeStruct((seq_len, d_model), pe2d.dtype),
        scratch_types=[
            pltpu.VMEM((2, chunk, d_model), pe2d.dtype),
            pltpu.SemaphoreType.DMA,
            pltpu.SemaphoreType.DMA,
        ],
    )
    def copy_kernel(pe_hbm, out_hbm, buf, in_sem, out_sem):
        wid = lax.axis_index("s") * _NUM_CORES + lax.axis_index("c")
        base = wid * rows_per_w

        loads = []
        stores = []
        for i in range(n_chunks):
            lo = base + i * chunk
            loads.append(pltpu.make_async_copy(
                pe_hbm.at[pl.ds(lo, chunk)], buf.at[i % 2], in_sem))
            stores.append(pltpu.make_async_copy(
                buf.at[i % 2], out_hbm.at[pl.ds(lo, chunk)], out_sem))

        loads[0].start()
        loads[1].start()
        for i in range(n_chunks):
            loads[i].wait()
            stores[i].start()
            if i + 2 < n_chunks:
                stores[i].wait()
                loads[i + 2].start()
        stores[n_chunks - 2].wait()
        stores[n_chunks - 1].wait()

    return copy_kernel(pe2d)


@functools.partial(jax.jit, static_argnums=(1, 2, 3))
def _sc_direct_copy(pe2d, seq_len, d_model, n_chunks=1):
    rows_per_w = seq_len // _NUM_WORKERS
    chunk = rows_per_w // n_chunks
    mesh = plsc.VectorSubcoreMesh(
        core_axis_name="c", subcore_axis_name="s")

    @functools.partial(
        pl.kernel,
        mesh=mesh,
        out_type=jax.ShapeDtypeStruct((seq_len, d_model), pe2d.dtype),
        scratch_types=[pltpu.SemaphoreType.DMA],
    )
    def copy_kernel(pe_hbm, out_hbm, sem):
        wid = lax.axis_index("s") * _NUM_CORES + lax.axis_index("c")
        base = wid * rows_per_w
        copies = []
        for i in range(n_chunks):
            lo = base + i * chunk
            copies.append(pltpu.make_async_copy(
                pe_hbm.at[pl.ds(lo, chunk)], out_hbm.at[pl.ds(lo, chunk)],
                sem))
        for c in copies:
            c.start()
        for c in copies:
            c.wait()

    return copy_kernel(pe2d)


def _tc_direct_copy(pe2d, seq_len, d_model, n_chunks=8):
    chunk = seq_len // n_chunks

    def body(src_hbm, out_hbm, sem):
        copies = []
        for i in range(n_chunks):
            lo = i * chunk
            copies.append(pltpu.make_async_copy(
                src_hbm.at[pl.ds(lo, chunk)], out_hbm.at[pl.ds(lo, chunk)],
                sem))
        for c in copies:
            c.start()
        for c in copies:
            c.wait()

    return pl.pallas_call(
        body,
        in_specs=[pl.BlockSpec(memory_space=pltpu.MemorySpace.HBM)],
        out_specs=pl.BlockSpec(memory_space=pltpu.MemorySpace.HBM),
        out_shape=jax.ShapeDtypeStruct((seq_len, d_model), pe2d.dtype),
        scratch_shapes=[pltpu.SemaphoreType.DMA],
    )(pe2d)


def _tc_dma_copy(pe2d, seq_len, d_model, n_chunks=8, nbuf=4):
    chunk = seq_len // n_chunks

    assert nbuf == n_chunks

    def body(src_hbm, out_hbm, buf, in_sems, out_sem):
        loads = []
        stores = []
        for i in range(n_chunks):
            lo = i * chunk
            loads.append(pltpu.make_async_copy(
                src_hbm.at[pl.ds(lo, chunk)], buf.at[i], in_sems.at[i]))
            stores.append(pltpu.make_async_copy(
                buf.at[i], out_hbm.at[pl.ds(lo, chunk)], out_sem))
        for i in range(n_chunks):
            loads[i].start()
        for i in range(n_chunks):
            loads[i].wait()
            stores[i].start()
        for i in range(n_chunks):
            stores[i].wait()

    return pl.pallas_call(
        body,
        in_specs=[pl.BlockSpec(memory_space=pltpu.MemorySpace.HBM)],
        out_specs=pl.BlockSpec(memory_space=pltpu.MemorySpace.HBM),
        out_shape=jax.ShapeDtypeStruct((seq_len, d_model), pe2d.dtype),
        scratch_shapes=[
            pltpu.VMEM((nbuf, chunk, d_model), pe2d.dtype),
            pltpu.SemaphoreType.DMA((n_chunks,)),
            pltpu.SemaphoreType.DMA,
        ],
    )(pe2d)


def kernel(a_x, pe):
    seq_len = a_x.shape[1]
    _, max_len, d_model = pe.shape
    out = _tc_dma_copy(pe.reshape(max_len, d_model), seq_len, d_model,
                       n_chunks=4, nbuf=4)
    return out.reshape(1, seq_len, d_model)
